# baseline (speedup 1.0000x reference)
import functools

import numpy as np
import jax
import jax.numpy as jnp
from jax import lax
from jax.experimental import pallas as pl
from jax.experimental.pallas import tpu as pltpu
from jax.experimental.pallas import tpu_sc as plsc

_N = 10000
_E = 320000
_D = 128
_DE = 16
_H = 4
_DH = 32
_TD = 32
_NC = 2
_STEPS = 16
_DT = 1.0 / _STEPS
_SCALE = 1.0 / float(np.sqrt(_DH))

_SC_CORES = 2
_SC_SUB = 16
_NW = _SC_CORES * _SC_SUB
_BG = 128
_NBLK = 79
_CHUNK = _BG * _NBLK
_EP = _CHUNK * _NW
_NPAD = 10240
_SLAB = _NPAD // _SC_SUB

_EB = 1024
_NB = 1000

_F32 = jnp.float32


def _mesh():
    return plsc.VectorSubcoreMesh(
        core_axis_name="c", subcore_axis_name="s",
        num_cores=_SC_CORES, num_subcores=_SC_SUB)


def _gather_body(qtab, kvtab, dsti, srci, qd_out, kvs_out,
                 id0, id1, is0, is1, qr, kv,
                 sdi0, sdi1, ssi0, ssi1,
                 sq0, sq1, sk0, sk1, swq0, swq1, swk0, swk1):
    cid = lax.axis_index("c")
    sid = lax.axis_index("s")
    wid = sid * _SC_CORES + cid
    base = wid * _CHUNK
    idb = (id0, id1)
    isb = (is0, is1)
    semdi = (sdi0, sdi1)
    semsi = (ssi0, ssi1)
    semq = (sq0, sq1)
    semk = (sk0, sk1)
    semwq = (swq0, swq1)
    semwk = (swk0, swk1)

    def fire_idx(g, j):
        off = base + g * _BG
        pltpu.async_copy(dsti.at[pl.ds(off, _BG)], idb[j], semdi[j])
        pltpu.async_copy(srci.at[pl.ds(off, _BG)], isb[j], semsi[j])

    def wait_idx(g, j):
        off = base + g * _BG
        pltpu.make_async_copy(dsti.at[pl.ds(off, _BG)], idb[j], semdi[j]).wait()
        pltpu.make_async_copy(srci.at[pl.ds(off, _BG)], isb[j], semsi[j]).wait()

    def fire_gather(j):
        pltpu.async_copy(qtab.at[idb[j]], qr.at[j], semq[j])
        pltpu.async_copy(kvtab.at[isb[j]], kv.at[j], semk[j])

    def wait_gather(j):
        pltpu.make_async_copy(qtab.at[idb[j]], qr.at[j], semq[j]).wait()
        pltpu.make_async_copy(kvtab.at[isb[j]], kv.at[j], semk[j]).wait()

    def fire_wb(g, j):
        off = base + g * _BG
        pltpu.async_copy(qr.at[j], qd_out.at[pl.ds(off, _BG)], semwq[j])
        pltpu.async_copy(kv.at[j], kvs_out.at[pl.ds(off, _BG)], semwk[j])

    def wait_wb(g, j):
        off = base + g * _BG
        pltpu.make_async_copy(qr.at[j], qd_out.at[pl.ds(off, _BG)], semwq[j]).wait()
        pltpu.make_async_copy(kv.at[j], kvs_out.at[pl.ds(off, _BG)], semwk[j]).wait()

    fire_idx(0, 0)
    fire_idx(1, 1)
    wait_idx(0, 0)
    fire_gather(0)

    def outer(t, carry):
        for j in (0, 1):
            g = 2 * t + j

            @pl.when(g >= 1)
            def _():
                wait_wb(g - 1, 1 - j)

            wait_gather(j)

            @pl.when(g + 2 < _NBLK)
            def _():
                fire_idx(g + 2, j)

            @pl.when(g + 1 < _NBLK)
            def _():
                wait_idx(g + 1, 1 - j)
                fire_gather(1 - j)

            fire_wb(g, j)
        return carry

    lax.fori_loop(0, (_NBLK - 1) // 2, outer, 0)
    g = _NBLK - 1
    wait_wb(g - 1, 1)
    wait_gather(0)
    fire_wb(g, 0)
    wait_wb(g, 0)


@functools.lru_cache(maxsize=None)
def _get_gather():
    return pl.kernel(
        _gather_body,
        out_type=[jax.ShapeDtypeStruct((_EP, _D), _F32),
                  jax.ShapeDtypeStruct((_EP, 2 * _D), _F32)],
        mesh=_mesh(),
        scratch_types=[pltpu.VMEM((_BG,), jnp.int32),
                       pltpu.VMEM((_BG,), jnp.int32),
                       pltpu.VMEM((_BG,), jnp.int32),
                       pltpu.VMEM((_BG,), jnp.int32),
                       pltpu.VMEM((2, _BG, _D), _F32),
                       pltpu.VMEM((2, _BG, 2 * _D), _F32)]
                      + [pltpu.SemaphoreType.DMA] * 12,
    )


def _scatter1_body(m, dsti, zrow, pout, i0, i1, mb, acc,
                   si0, si1, sl0, sl1, ss0, ss1):
    cid = lax.axis_index("c")
    sid = lax.axis_index("s")
    wid = sid * _SC_CORES + cid
    r0 = sid * _SLAB
    base = wid * _CHUNK
    ib = (i0, i1)
    semi = (si0, si1)
    semld = (sl0, sl1)
    semsc = (ss0, ss1)

    pltpu.sync_copy(zrow, acc.at[pl.ds(r0, _SLAB)])
    plsc.subcore_barrier()

    def fire_load(g, j):
        off = base + g * _BG
        pltpu.async_copy(dsti.at[pl.ds(off, _BG)], ib[j], semi[j])
        pltpu.async_copy(m.at[pl.ds(off, _BG)], mb.at[j], semld[j])

    def wait_load(g, j):
        off = base + g * _BG
        pltpu.make_async_copy(dsti.at[pl.ds(off, _BG)], ib[j], semi[j]).wait()
        pltpu.make_async_copy(m.at[pl.ds(off, _BG)], mb.at[j], semld[j]).wait()

    def fire_sc(j):
        pltpu.async_copy(mb.at[j], acc.at[ib[j]], semsc[j], add=True)

    def wait_sc(j):
        pltpu.make_async_copy(mb.at[j], acc.at[ib[j]], semsc[j]).wait()

    fire_load(0, 0)

    def outer(t, carry):
        for j in (0, 1):
            g = 2 * t + j

            @pl.when(g >= 1)
            def _():
                wait_sc(1 - j)

            @pl.when(g + 1 < _NBLK)
            def _():
                fire_load(g + 1, 1 - j)

            wait_load(g, j)
            fire_sc(j)
        return carry

    lax.fori_loop(0, (_NBLK - 1) // 2, outer, 0)
    g = _NBLK - 1
    wait_sc(1)
    wait_load(g, 0)
    fire_sc(0)
    wait_sc(0)

    plsc.subcore_barrier()
    o0 = cid * _NPAD + r0
    pltpu.sync_copy(acc.at[pl.ds(r0, _SLAB)], pout.at[pl.ds(o0, _SLAB)])


@functools.lru_cache(maxsize=None)
def _get_scatter1():
    return pl.kernel(
        _scatter1_body,
        out_type=[jax.ShapeDtypeStruct((_SC_CORES * _NPAD, _D), _F32)],
        mesh=_mesh(),
        scratch_types=[pltpu.VMEM((_BG,), jnp.int32),
                       pltpu.VMEM((_BG,), jnp.int32),
                       pltpu.VMEM((2, _BG, _D), _F32),
                       pltpu.MemorySpace.VMEM_SHARED((_NPAD, _D), _F32)]
                      + [pltpu.SemaphoreType.DMA] * 6,
    )


def _node_prep_body(nf, win, bin_, wq, wkv, h_out, q_out, kv_out):
    h = jnp.dot(nf[...], win[...], preferred_element_type=_F32) + bin_[...]
    h_out[...] = h
    q_out[...] = jnp.dot(h, wq[...], preferred_element_type=_F32)
    kv_out[...] = jnp.dot(h, wkv[...], preferred_element_type=_F32)


_node_prep = pl.pallas_call(
    _node_prep_body,
    grid=(_N // _NB,),
    in_specs=[pl.BlockSpec((_NB, _D), lambda i: (i, 0)),
              pl.BlockSpec((_D, _D), lambda i: (0, 0)),
              pl.BlockSpec((1, _D), lambda i: (0, 0)),
              pl.BlockSpec((_D, _D), lambda i: (0, 0)),
              pl.BlockSpec((_D, 2 * _D), lambda i: (0, 0))],
    out_specs=[pl.BlockSpec((_NB, _D), lambda i: (i, 0)),
               pl.BlockSpec((_NB, _D), lambda i: (i, 0)),
               pl.BlockSpec((_NB, 2 * _D), lambda i: (i, 0))],
    out_shape=[jax.ShapeDtypeStruct((_N, _D), _F32),
               jax.ShapeDtypeStruct((_N, _D), _F32),
               jax.ShapeDtypeStruct((_N, 2 * _D), _F32)],
)


def _edge_prep_body(ef, et, freq, wks, wvs, k_out, v_out):
    tau = 1.0 - et[...]
    ang = tau * freq[...]
    sk = jnp.concatenate([ef[...], jnp.sin(ang), jnp.cos(ang)], axis=1)
    k_out[...] = jnp.dot(sk, wks[...], preferred_element_type=_F32)
    v_out[...] = jnp.dot(sk, wvs[...], preferred_element_type=_F32)


_edge_prep = pl.pallas_call(
    _edge_prep_body,
    grid=(_EP // _EB,),
    in_specs=[pl.BlockSpec((_EB, _DE), lambda i: (i, 0)),
              pl.BlockSpec((_EB, 1), lambda i: (i, 0)),
              pl.BlockSpec((1, _TD // 2), lambda i: (0, 0)),
              pl.BlockSpec((_DE + _TD, _D), lambda i: (0, 0)),
              pl.BlockSpec((_DE + _TD, _D), lambda i: (0, 0))],
    out_specs=[pl.BlockSpec((_EB, _D), lambda i: (i, 0)),
               pl.BlockSpec((_EB, _D), lambda i: (i, 0))],
    out_shape=[jax.ShapeDtypeStruct((_EP, _D), _F32),
               jax.ShapeDtypeStruct((_EP, _D), _F32)],
)


def _edge_math_body(qd, kvs, kst, vst, m_out, e_out):
    q = qd[...]
    k = kvs[:, :_D] + kst[...]
    v = kvs[:, _D:] + vst[...]
    cols = []
    exs = []
    for hh in range(_H):
        sl = slice(hh * _DH, (hh + 1) * _DH)
        s = jnp.sum(q[:, sl] * k[:, sl], axis=1, keepdims=True) * _SCALE
        ex = jnp.exp(s)
        exs.append(jnp.broadcast_to(ex, (_EB, _DH)))
        cols.append(ex * v[:, sl])
    m_out[...] = jnp.concatenate(cols, axis=1)
    e_out[...] = jnp.concatenate(exs, axis=1)


_edge_math = pl.pallas_call(
    _edge_math_body,
    grid=(_EP // _EB,),
    in_specs=[pl.BlockSpec((_EB, _D), lambda i: (i, 0)),
              pl.BlockSpec((_EB, 2 * _D), lambda i: (i, 0)),
              pl.BlockSpec((_EB, _D), lambda i: (i, 0)),
              pl.BlockSpec((_EB, _D), lambda i: (i, 0))],
    out_specs=[pl.BlockSpec((_EB, _D), lambda i: (i, 0)),
               pl.BlockSpec((_EB, _D), lambda i: (i, 0))],
    out_shape=[jax.ShapeDtypeStruct((_EP, _D), _F32),
               jax.ShapeDtypeStruct((_EP, _D), _F32)],
)


def _update_body(p0, p1, e0, e1, h_in, wo, bo_, wq, wkv, h_out, q_out, kv_out):
    num = p0[0] + p1[0]
    den = e0[0] + e1[0]
    parts = []
    for hh in range(_H):
        d = den[:, hh * _DH:hh * _DH + 1] + 1e-16
        parts.append(num[:, hh * _DH:(hh + 1) * _DH] / d)
    agg = jnp.concatenate(parts, axis=1)
    dh = jnp.tanh(jnp.dot(agg, wo[...], preferred_element_type=_F32) + bo_[...])
    hn = h_in[...] + _DT * dh
    h_out[...] = hn
    q_out[...] = jnp.dot(hn, wq[...], preferred_element_type=_F32)
    kv_out[...] = jnp.dot(hn, wkv[...], preferred_element_type=_F32)


_update = pl.pallas_call(
    _update_body,
    grid=(_N // _NB,),
    in_specs=[pl.BlockSpec((1, _NB, _D), lambda i: (0, i, 0)),
              pl.BlockSpec((1, _NB, _D), lambda i: (1, i, 0)),
              pl.BlockSpec((1, _NB, _D), lambda i: (0, i, 0)),
              pl.BlockSpec((1, _NB, _D), lambda i: (1, i, 0)),
              pl.BlockSpec((_NB, _D), lambda i: (i, 0)),
              pl.BlockSpec((_D, _D), lambda i: (0, 0)),
              pl.BlockSpec((1, _D), lambda i: (0, 0)),
              pl.BlockSpec((_D, _D), lambda i: (0, 0)),
              pl.BlockSpec((_D, 2 * _D), lambda i: (0, 0))],
    out_specs=[pl.BlockSpec((_NB, _D), lambda i: (i, 0)),
               pl.BlockSpec((_NB, _D), lambda i: (i, 0)),
               pl.BlockSpec((_NB, 2 * _D), lambda i: (i, 0))],
    out_shape=[jax.ShapeDtypeStruct((_N, _D), _F32),
               jax.ShapeDtypeStruct((_N, _D), _F32),
               jax.ShapeDtypeStruct((_N, 2 * _D), _F32)],
)


def _cls_body(h_in, wc1, bc1_, wc2, bc2_, lg_out, pr_out):
    hd = jnp.dot(h_in[...], wc1[...], preferred_element_type=_F32) + bc1_[...]
    hd = hd * jax.nn.sigmoid(hd)
    lg = jnp.dot(hd, wc2[...], preferred_element_type=_F32) + bc2_[...]
    lg_out[...] = lg
    mx = jnp.max(lg, axis=1, keepdims=True)
    e = jnp.exp(lg - mx)
    pr_out[...] = e / jnp.sum(e, axis=1, keepdims=True)


_cls = pl.pallas_call(
    _cls_body,
    grid=(_N // _NB,),
    in_specs=[pl.BlockSpec((_NB, _D), lambda i: (i, 0)),
              pl.BlockSpec((_D, _D), lambda i: (0, 0)),
              pl.BlockSpec((1, _D), lambda i: (0, 0)),
              pl.BlockSpec((_D, _NC), lambda i: (0, 0)),
              pl.BlockSpec((1, _NC), lambda i: (0, 0))],
    out_specs=[pl.BlockSpec((_NB, _NC), lambda i: (i, 0)),
               pl.BlockSpec((_NB, _NC), lambda i: (i, 0))],
    out_shape=[jax.ShapeDtypeStruct((_N, _NC), _F32),
               jax.ShapeDtypeStruct((_N, _NC), _F32)],
)


def kernel(node_features, edge_index, edge_features, edge_times, W_in, b_in,
           time_freq, Wq, Wk, Wv, Wo, bo, Wc1, bc1, Wc2, bc2):
    pad = _EP - _E
    src = jnp.pad(edge_index[0], (0, pad))
    dstg = jnp.pad(edge_index[1], (0, pad))
    dump = _N + (jnp.arange(pad, dtype=jnp.int32) % (_NPAD - _N))
    dsts = jnp.concatenate([edge_index[1], dump])
    wkv = jnp.concatenate([Wk[:_D], Wv[:_D]], axis=1)
    wks = Wk[_D:]
    wvs = Wv[_D:]
    b_in2 = b_in.reshape(1, _D)
    bo2 = bo.reshape(1, _D)
    bc12 = bc1.reshape(1, _D)
    bc22 = bc2.reshape(1, _NC)
    et2 = jnp.pad(edge_times, (0, pad)).reshape(_EP, 1)
    efp = jnp.pad(edge_features, ((0, pad), (0, 0)))
    freq2 = time_freq.reshape(1, _TD // 2)
    zrow = jnp.zeros((_SLAB, _D), _F32)

    h, qtab, kvtab = _node_prep(node_features, W_in, b_in2, Wq, wkv)
    kst, vst = _edge_prep(efp, et2, freq2, wks, wvs)
    gather_k = _get_gather()
    scatter_k = _get_scatter1()
    for _ in range(_STEPS):
        qd, kvs = gather_k(qtab, kvtab, dstg, src)
        m, mex = _edge_math(qd, kvs, kst, vst)
        (pflat,) = scatter_k(m, dsts, zrow)
        (eflat,) = scatter_k(mex, dsts, zrow)
        pout = pflat.reshape(_SC_CORES, _NPAD, _D)
        eout = eflat.reshape(_SC_CORES, _NPAD, _D)
        h, qtab, kvtab = _update(pout, pout, eout, eout, h, Wo, bo2, Wq, wkv)
    logits, probs = _cls(h, Wc1, bc12, Wc2, bc22)
    return h, logits, probs[:, 0]


# speedup vs baseline: 1.1446x; 1.1446x over previous
import functools

import numpy as np
import jax
import jax.numpy as jnp
from jax import lax
from jax.experimental import pallas as pl
from jax.experimental.pallas import tpu as pltpu
from jax.experimental.pallas import tpu_sc as plsc

_N = 10000
_E = 320000
_D = 128
_DE = 16
_H = 4
_DH = 32
_TD = 32
_NC = 2
_STEPS = 16
_DT = 1.0 / _STEPS
_SCALE = 1.0 / float(np.sqrt(_DH))

_SC_CORES = 2
_SC_SUB = 16
_NW = _SC_CORES * _SC_SUB
_BG = 80
_NBLK = 125
_CHUNK = _BG * _NBLK
_EP = _CHUNK * _NW
_NPAD = 10240
_SLAB = _NPAD // _SC_SUB

_EB = 1000
_NB = 1000

_F32 = jnp.float32


def _mesh():
    return plsc.VectorSubcoreMesh(
        core_axis_name="c", subcore_axis_name="s",
        num_cores=_SC_CORES, num_subcores=_SC_SUB)


def _gather_body(qtab, kvtab, dsti, srci, qd_out, kvs_out,
                 id0, id1, id2, is0, is1, is2, qr, kv,
                 sdi0, sdi1, sdi2, ssi0, ssi1, ssi2,
                 sq0, sq1, sq2, sk0, sk1, sk2,
                 swq0, swq1, swq2, swk0, swk1, swk2):
    cid = lax.axis_index("c")
    sid = lax.axis_index("s")
    wid = sid * _SC_CORES + cid
    base = wid * _CHUNK
    idb = (id0, id1, id2)
    isb = (is0, is1, is2)
    semdi = (sdi0, sdi1, sdi2)
    semsi = (ssi0, ssi1, ssi2)
    semq = (sq0, sq1, sq2)
    semk = (sk0, sk1, sk2)
    semwq = (swq0, swq1, swq2)
    semwk = (swk0, swk1, swk2)

    def fire_idx(g, j):
        off = base + g * _BG
        pltpu.async_copy(dsti.at[pl.ds(off, _BG)], idb[j], semdi[j])
        pltpu.async_copy(srci.at[pl.ds(off, _BG)], isb[j], semsi[j])

    def wait_idx(g, j):
        off = base + g * _BG
        pltpu.make_async_copy(dsti.at[pl.ds(off, _BG)], idb[j], semdi[j]).wait()
        pltpu.make_async_copy(srci.at[pl.ds(off, _BG)], isb[j], semsi[j]).wait()

    def fire_gather(j):
        pltpu.async_copy(qtab.at[idb[j]], qr.at[j], semq[j])
        pltpu.async_copy(kvtab.at[isb[j]], kv.at[j], semk[j])

    def wait_gather(j):
        pltpu.make_async_copy(qtab.at[idb[j]], qr.at[j], semq[j]).wait()
        pltpu.make_async_copy(kvtab.at[isb[j]], kv.at[j], semk[j]).wait()

    def fire_wb(g, j):
        off = base + g * _BG
        pltpu.async_copy(qr.at[j], qd_out.at[pl.ds(off, _BG)], semwq[j])
        pltpu.async_copy(kv.at[j], kvs_out.at[pl.ds(off, _BG)], semwk[j])

    def wait_wb(g, j):
        off = base + g * _BG
        pltpu.make_async_copy(qr.at[j], qd_out.at[pl.ds(off, _BG)], semwq[j]).wait()
        pltpu.make_async_copy(kv.at[j], kvs_out.at[pl.ds(off, _BG)], semwk[j]).wait()

    fire_idx(0, 0)
    fire_idx(1, 1)
    wait_idx(0, 0)
    fire_gather(0)
    fire_idx(2, 2)
    wait_idx(1, 1)
    fire_gather(1)

    def step(g, j):
        wait_gather(j)
        fire_wb(g, j)

        @pl.when(g >= 1)
        def _():
            wait_wb(g - 1, (j + 2) % 3)

        @pl.when(g + 3 < _NBLK)
        def _():
            fire_idx(g + 3, j)

        @pl.when(g + 2 < _NBLK)
        def _():
            wait_idx(g + 2, (j + 2) % 3)
            fire_gather((j + 2) % 3)

    def outer(t, carry):
        for j in (0, 1, 2):
            step(3 * t + j, j)
        return carry

    lax.fori_loop(0, (_NBLK - 2) // 3, outer, 0)
    for g in (_NBLK - 2, _NBLK - 1):
        step(g, g % 3)
    wait_wb(_NBLK - 1, (_NBLK - 1) % 3)


@functools.lru_cache(maxsize=None)
def _get_gather():
    return pl.kernel(
        _gather_body,
        out_type=[jax.ShapeDtypeStruct((_EP, _D), _F32),
                  jax.ShapeDtypeStruct((_EP, 2 * _D), _F32)],
        mesh=_mesh(),
        scratch_types=[pltpu.VMEM((_BG,), jnp.int32)] * 6
                      + [pltpu.VMEM((3, _BG, _D), _F32),
                         pltpu.VMEM((3, _BG, 2 * _D), _F32)]
                      + [pltpu.SemaphoreType.DMA] * 18,
    )


def _scatter1_body(m, dsti, zrow, pout, i0, i1, i2, i3, mb, acc,
                   si0, si1, si2, si3, sl0, sl1, sl2, sl3,
                   ss0, ss1, ss2, ss3):
    cid = lax.axis_index("c")
    sid = lax.axis_index("s")
    wid = sid * _SC_CORES + cid
    r0 = sid * _SLAB
    base = wid * _CHUNK
    ib = (i0, i1, i2, i3)
    semi = (si0, si1, si2, si3)
    semld = (sl0, sl1, sl2, sl3)
    semsc = (ss0, ss1, ss2, ss3)

    pltpu.sync_copy(zrow, acc.at[pl.ds(r0, _SLAB)])
    plsc.subcore_barrier()

    def fire_load(g, j):
        off = base + g * _BG
        pltpu.async_copy(dsti.at[pl.ds(off, _BG)], ib[j], semi[j])
        pltpu.async_copy(m.at[pl.ds(off, _BG)], mb.at[j], semld[j])

    def wait_load(g, j):
        off = base + g * _BG
        pltpu.make_async_copy(dsti.at[pl.ds(off, _BG)], ib[j], semi[j]).wait()
        pltpu.make_async_copy(m.at[pl.ds(off, _BG)], mb.at[j], semld[j]).wait()

    def fire_sc(j):
        pltpu.async_copy(mb.at[j], acc.at[ib[j]], semsc[j], add=True)

    def wait_sc(j):
        pltpu.make_async_copy(mb.at[j], acc.at[ib[j]], semsc[j]).wait()

    fire_load(0, 0)
    fire_load(1, 1)

    def step(g, j):
        wait_load(g, j)
        fire_sc(j)

        @pl.when(g >= 2)
        def _():
            wait_sc((j + 2) % 4)

        @pl.when(g + 2 < _NBLK)
        def _():
            fire_load(g + 2, (j + 2) % 4)

    def outer(t, carry):
        for j in (0, 1, 2, 3):
            step(4 * t + j, j)
        return carry

    lax.fori_loop(0, (_NBLK - 1) // 4, outer, 0)
    g0 = ((_NBLK - 1) // 4) * 4
    for g in range(g0, _NBLK):
        step(g, g % 4)
    wait_sc((_NBLK - 2) % 4)
    wait_sc((_NBLK - 1) % 4)

    plsc.subcore_barrier()
    o0 = cid * _NPAD + r0
    pltpu.sync_copy(acc.at[pl.ds(r0, _SLAB)], pout.at[pl.ds(o0, _SLAB)])


@functools.lru_cache(maxsize=None)
def _get_scatter1():
    return pl.kernel(
        _scatter1_body,
        out_type=[jax.ShapeDtypeStruct((_SC_CORES * _NPAD, _D), _F32)],
        mesh=_mesh(),
        scratch_types=[pltpu.VMEM((_BG,), jnp.int32)] * 4
                      + [pltpu.VMEM((4, _BG, _D), _F32),
                         pltpu.MemorySpace.VMEM_SHARED((_NPAD, _D), _F32)]
                      + [pltpu.SemaphoreType.DMA] * 12,
    )


def _node_prep_body(nf, win, bin_, wq, wkv, h_out, q_out, kv_out):
    h = jnp.dot(nf[...], win[...], preferred_element_type=_F32) + bin_[...]
    h_out[...] = h
    q_out[...] = jnp.dot(h, wq[...], preferred_element_type=_F32)
    kv_out[...] = jnp.dot(h, wkv[...], preferred_element_type=_F32)


_node_prep = pl.pallas_call(
    _node_prep_body,
    grid=(_N // _NB,),
    in_specs=[pl.BlockSpec((_NB, _D), lambda i: (i, 0)),
              pl.BlockSpec((_D, _D), lambda i: (0, 0)),
              pl.BlockSpec((1, _D), lambda i: (0, 0)),
              pl.BlockSpec((_D, _D), lambda i: (0, 0)),
              pl.BlockSpec((_D, 2 * _D), lambda i: (0, 0))],
    out_specs=[pl.BlockSpec((_NB, _D), lambda i: (i, 0)),
               pl.BlockSpec((_NB, _D), lambda i: (i, 0)),
               pl.BlockSpec((_NB, 2 * _D), lambda i: (i, 0))],
    out_shape=[jax.ShapeDtypeStruct((_N, _D), _F32),
               jax.ShapeDtypeStruct((_N, _D), _F32),
               jax.ShapeDtypeStruct((_N, 2 * _D), _F32)],
)


def _edge_prep_body(ef, et, freq, wks, wvs, k_out, v_out):
    tau = 1.0 - et[...]
    ang = tau * freq[...]
    sk = jnp.concatenate([ef[...], jnp.sin(ang), jnp.cos(ang)], axis=1)
    k_out[...] = jnp.dot(sk, wks[...], preferred_element_type=_F32)
    v_out[...] = jnp.dot(sk, wvs[...], preferred_element_type=_F32)


_edge_prep = pl.pallas_call(
    _edge_prep_body,
    grid=(_EP // _EB,),
    in_specs=[pl.BlockSpec((_EB, _DE), lambda i: (i, 0)),
              pl.BlockSpec((_EB, 1), lambda i: (i, 0)),
              pl.BlockSpec((1, _TD // 2), lambda i: (0, 0)),
              pl.BlockSpec((_DE + _TD, _D), lambda i: (0, 0)),
              pl.BlockSpec((_DE + _TD, _D), lambda i: (0, 0))],
    out_specs=[pl.BlockSpec((_EB, _D), lambda i: (i, 0)),
               pl.BlockSpec((_EB, _D), lambda i: (i, 0))],
    out_shape=[jax.ShapeDtypeStruct((_EP, _D), _F32),
               jax.ShapeDtypeStruct((_EP, _D), _F32)],
)


def _edge_math_body(qd, kvs, kst, vst, m_out, e_out):
    q = qd[...]
    k = kvs[:, :_D] + kst[...]
    v = kvs[:, _D:] + vst[...]
    cols = []
    exs = []
    for hh in range(_H):
        sl = slice(hh * _DH, (hh + 1) * _DH)
        s = jnp.sum(q[:, sl] * k[:, sl], axis=1, keepdims=True) * _SCALE
        ex = jnp.exp(s)
        exs.append(jnp.broadcast_to(ex, (_EB, _DH)))
        cols.append(ex * v[:, sl])
    m_out[...] = jnp.concatenate(cols, axis=1)
    e_out[...] = jnp.concatenate(exs, axis=1)


_edge_math = pl.pallas_call(
    _edge_math_body,
    grid=(_EP // _EB,),
    in_specs=[pl.BlockSpec((_EB, _D), lambda i: (i, 0)),
              pl.BlockSpec((_EB, 2 * _D), lambda i: (i, 0)),
              pl.BlockSpec((_EB, _D), lambda i: (i, 0)),
              pl.BlockSpec((_EB, _D), lambda i: (i, 0))],
    out_specs=[pl.BlockSpec((_EB, _D), lambda i: (i, 0)),
               pl.BlockSpec((_EB, _D), lambda i: (i, 0))],
    out_shape=[jax.ShapeDtypeStruct((_EP, _D), _F32),
               jax.ShapeDtypeStruct((_EP, _D), _F32)],
)


def _update_body(p0, p1, e0, e1, h_in, wo, bo_, wq, wkv, h_out, q_out, kv_out):
    num = p0[0] + p1[0]
    den = e0[0] + e1[0]
    parts = []
    for hh in range(_H):
        d = den[:, hh * _DH:hh * _DH + 1] + 1e-16
        parts.append(num[:, hh * _DH:(hh + 1) * _DH] / d)
    agg = jnp.concatenate(parts, axis=1)
    dh = jnp.tanh(jnp.dot(agg, wo[...], preferred_element_type=_F32) + bo_[...])
    hn = h_in[...] + _DT * dh
    h_out[...] = hn
    q_out[...] = jnp.dot(hn, wq[...], preferred_element_type=_F32)
    kv_out[...] = jnp.dot(hn, wkv[...], preferred_element_type=_F32)


_update = pl.pallas_call(
    _update_body,
    grid=(_N // _NB,),
    in_specs=[pl.BlockSpec((1, _NB, _D), lambda i: (0, i, 0)),
              pl.BlockSpec((1, _NB, _D), lambda i: (1, i, 0)),
              pl.BlockSpec((1, _NB, _D), lambda i: (0, i, 0)),
              pl.BlockSpec((1, _NB, _D), lambda i: (1, i, 0)),
              pl.BlockSpec((_NB, _D), lambda i: (i, 0)),
              pl.BlockSpec((_D, _D), lambda i: (0, 0)),
              pl.BlockSpec((1, _D), lambda i: (0, 0)),
              pl.BlockSpec((_D, _D), lambda i: (0, 0)),
              pl.BlockSpec((_D, 2 * _D), lambda i: (0, 0))],
    out_specs=[pl.BlockSpec((_NB, _D), lambda i: (i, 0)),
               pl.BlockSpec((_NB, _D), lambda i: (i, 0)),
               pl.BlockSpec((_NB, 2 * _D), lambda i: (i, 0))],
    out_shape=[jax.ShapeDtypeStruct((_N, _D), _F32),
               jax.ShapeDtypeStruct((_N, _D), _F32),
               jax.ShapeDtypeStruct((_N, 2 * _D), _F32)],
)


def _cls_body(h_in, wc1, bc1_, wc2, bc2_, lg_out, pr_out):
    hd = jnp.dot(h_in[...], wc1[...], preferred_element_type=_F32) + bc1_[...]
    hd = hd * jax.nn.sigmoid(hd)
    lg = jnp.dot(hd, wc2[...], preferred_element_type=_F32) + bc2_[...]
    lg_out[...] = lg
    mx = jnp.max(lg, axis=1, keepdims=True)
    e = jnp.exp(lg - mx)
    pr_out[...] = e / jnp.sum(e, axis=1, keepdims=True)


_cls = pl.pallas_call(
    _cls_body,
    grid=(_N // _NB,),
    in_specs=[pl.BlockSpec((_NB, _D), lambda i: (i, 0)),
              pl.BlockSpec((_D, _D), lambda i: (0, 0)),
              pl.BlockSpec((1, _D), lambda i: (0, 0)),
              pl.BlockSpec((_D, _NC), lambda i: (0, 0)),
              pl.BlockSpec((1, _NC), lambda i: (0, 0))],
    out_specs=[pl.BlockSpec((_NB, _NC), lambda i: (i, 0)),
               pl.BlockSpec((_NB, _NC), lambda i: (i, 0))],
    out_shape=[jax.ShapeDtypeStruct((_N, _NC), _F32),
               jax.ShapeDtypeStruct((_N, _NC), _F32)],
)


def kernel(node_features, edge_index, edge_features, edge_times, W_in, b_in,
           time_freq, Wq, Wk, Wv, Wo, bo, Wc1, bc1, Wc2, bc2):
    pad = _EP - _E
    src = jnp.pad(edge_index[0], (0, pad))
    dstg = jnp.pad(edge_index[1], (0, pad))
    dump = _N + (jnp.arange(pad, dtype=jnp.int32) % (_NPAD - _N))
    dsts = jnp.concatenate([edge_index[1], dump])
    wkv = jnp.concatenate([Wk[:_D], Wv[:_D]], axis=1)
    wks = Wk[_D:]
    wvs = Wv[_D:]
    b_in2 = b_in.reshape(1, _D)
    bo2 = bo.reshape(1, _D)
    bc12 = bc1.reshape(1, _D)
    bc22 = bc2.reshape(1, _NC)
    et2 = jnp.pad(edge_times, (0, pad)).reshape(_EP, 1)
    efp = jnp.pad(edge_features, ((0, pad), (0, 0)))
    freq2 = time_freq.reshape(1, _TD // 2)
    zrow = jnp.zeros((_SLAB, _D), _F32)

    h, qtab, kvtab = _node_prep(node_features, W_in, b_in2, Wq, wkv)
    kst, vst = _edge_prep(efp, et2, freq2, wks, wvs)
    gather_k = _get_gather()
    scatter_k = _get_scatter1()
    for _ in range(_STEPS):
        qd, kvs = gather_k(qtab, kvtab, dstg, src)
        m, mex = _edge_math(qd, kvs, kst, vst)
        (pflat,) = scatter_k(m, dsts, zrow)
        (eflat,) = scatter_k(mex, dsts, zrow)
        pout = pflat.reshape(_SC_CORES, _NPAD, _D)
        eout = eflat.reshape(_SC_CORES, _NPAD, _D)
        h, qtab, kvtab = _update(pout, pout, eout, eout, h, Wo, bo2, Wq, wkv)
    logits, probs = _cls(h, Wc1, bc12, Wc2, bc22)
    return h, logits, probs[:, 0]
